# baseline (speedup 1.0000x reference)
import functools

import jax
import jax.numpy as jnp
from jax import lax
from jax.experimental import pallas as pl
from jax.experimental.pallas import tpu as pltpu
from jax.experimental.pallas import tpu_sc as plsc

ITEM_NUM = 1000000
EMB_SIZE = 64
MAX_LEN = 200
BATCH = 4096

_INFO = plsc.get_sparse_core_info()
_NC = _INFO.num_cores
_NS = _INFO.num_subcores
_NW = _NC * _NS
_SEQ_PER_W = BATCH // _NW
_HALF = MAX_LEN // 2
_LANES = 16
_VPR = EMB_SIZE // _LANES


def _make_kernel():
    mesh = plsc.VectorSubcoreMesh(core_axis_name="c", subcore_axis_name="s")

    @functools.partial(
        pl.kernel,
        out_type=jax.ShapeDtypeStruct((BATCH * MAX_LEN, EMB_SIZE), jnp.float32),
        mesh=mesh,
        scratch_types=[
            pltpu.VMEM((_SEQ_PER_W, 2, _HALF), jnp.int32),
            pltpu.VMEM((MAX_LEN, EMB_SIZE), jnp.float32),
            pltpu.VMEM((MAX_LEN, EMB_SIZE), jnp.float32),
            pltpu.VMEM((MAX_LEN, EMB_SIZE), jnp.float32),
            pltpu.SemaphoreType.DMA,
            pltpu.SemaphoreType.DMA,
            pltpu.SemaphoreType.DMA,
            pltpu.SemaphoreType.DMA,
        ],
        compiler_params=pltpu.CompilerParams(use_tc_tiling_on_sc=False),
    )
    def emb_kernel(seqs_hbm, table_hbm, pos_hbm, out_hbm,
                   idx_all, pos_v, rows_a, rows_b, gsa, gsb, osa, osb):
        wid = lax.axis_index("s") * _NC + lax.axis_index("c")
        base_seq = wid * _SEQ_PER_W

        pltpu.sync_copy(pos_hbm, pos_v)
        pltpu.sync_copy(seqs_hbm.at[pl.ds(base_seq, _SEQ_PER_W)], idx_all)

        def gather_start(s, buf, sem):
            pltpu.async_copy(table_hbm.at[idx_all.at[s, 0]],
                             buf.at[pl.ds(0, _HALF)], sem)
            pltpu.async_copy(table_hbm.at[idx_all.at[s, 1]],
                             buf.at[pl.ds(_HALF, _HALF)], sem)

        def gather_wait(s, buf, sem):
            pltpu.make_async_copy(table_hbm.at[idx_all.at[s, 0]],
                                  buf.at[pl.ds(0, _HALF)], sem).wait()
            pltpu.make_async_copy(table_hbm.at[idx_all.at[s, 1]],
                                  buf.at[pl.ds(_HALF, _HALF)], sem).wait()

        def out_start(s, buf, sem):
            b = base_seq + s
            pltpu.async_copy(buf, out_hbm.at[pl.ds(b * MAX_LEN, MAX_LEN)], sem)

        def out_wait(s, buf, sem):
            b = base_seq + s
            pltpu.make_async_copy(
                buf, out_hbm.at[pl.ds(b * MAX_LEN, MAX_LEN)], sem).wait()

        def add_pos(buf):
            def add_row(r, c):
                for j in range(_VPR):
                    sl = pl.ds(j * _LANES, _LANES)
                    buf[r, sl] = buf[r, sl] + pos_v[r, sl]
                return c
            lax.fori_loop(0, MAX_LEN, add_row, 0, unroll=2)

        gather_start(0, rows_a, gsa)

        def body(i, carry):
            s0 = 2 * i
            s1 = s0 + 1

            @pl.when(i > 0)
            def _():
                out_wait(s1 - 2, rows_b, osb)
            gather_start(s1, rows_b, gsb)

            gather_wait(s0, rows_a, gsa)
            add_pos(rows_a)
            out_start(s0, rows_a, osa)

            @pl.when(i < _SEQ_PER_W // 2 - 1)
            def _():
                out_wait(s0, rows_a, osa)
                gather_start(s0 + 2, rows_a, gsa)

            gather_wait(s1, rows_b, gsb)
            add_pos(rows_b)
            out_start(s1, rows_b, osb)
            return carry

        lax.fori_loop(0, _SEQ_PER_W // 2, body, 0)
        out_wait(_SEQ_PER_W - 2, rows_a, osa)
        out_wait(_SEQ_PER_W - 1, rows_b, osb)

    return emb_kernel


_EMB_KERNEL = _make_kernel()


def kernel(batch_seqs, token_table, pos_table):
    seqs = batch_seqs.astype(jnp.int32).reshape(BATCH, 2, _HALF)
    out = _EMB_KERNEL(seqs, token_table, pos_table)
    return out.reshape(BATCH, MAX_LEN, EMB_SIZE)
